# baseline (speedup 1.0000x reference)
import functools

import jax
import jax.numpy as jnp
from jax import lax
from jax.experimental import pallas as pl
from jax.experimental.pallas import tpu as pltpu
from jax.experimental.pallas import tpu_sc as plsc


def _make_sc_gather(n_rows, d, rows_per_worker, num_cores):
    def body(table_hbm, idx_hbm, out_hbm, idx_v, rows_v, sem):
        wid = lax.axis_index("s") * num_cores + lax.axis_index("c")
        base = wid * rows_per_worker
        pltpu.sync_copy(idx_hbm.at[pl.ds(base, rows_per_worker)], idx_v)
        pltpu.async_copy(table_hbm.at[idx_v], rows_v, sem).wait()
        pltpu.sync_copy(rows_v, out_hbm.at[pl.ds(base, rows_per_worker)])

    return pl.kernel(
        body,
        out_type=jax.ShapeDtypeStruct((n_rows, d), jnp.int32),
        mesh=plsc.VectorSubcoreMesh(core_axis_name="c", subcore_axis_name="s"),
        scratch_types=[
            pltpu.VMEM((rows_per_worker,), jnp.int32),
            pltpu.VMEM((rows_per_worker, d), jnp.int32),
            pltpu.SemaphoreType.DMA,
        ],
    )


def _matmul_body(csz, n_out, add_bias, x_ref, w_ref, b_ref, o_ref):
    bc = x_ref.shape[0]
    bm = bc * csz
    nc = csz * n_out
    x3 = x_ref[...]
    xb = x3.reshape(bm, x3.shape[2]).astype(jnp.bfloat16)
    wb = w_ref[...]
    p = lax.dot_general(
        xb, wb, (((1,), (0,)), ((), ())), preferred_element_type=jnp.float32
    )
    rows = lax.broadcasted_iota(jnp.int32, p.shape, 0)
    lanes = lax.broadcasted_iota(jnp.int32, p.shape, 1)
    z = jnp.where((rows % csz) == (lanes // n_out), p, 0.0).astype(jnp.bfloat16)
    f = (
        lax.broadcasted_iota(jnp.int32, (nc, n_out), 0) % n_out
        == lax.broadcasted_iota(jnp.int32, (nc, n_out), 1)
    ).astype(jnp.bfloat16)
    r2 = lax.dot_general(
        z, f, (((1,), (0,)), ((), ())), preferred_element_type=jnp.float32
    )
    s = (
        lax.broadcasted_iota(jnp.int32, (bc, bm), 1) // csz
        == lax.broadcasted_iota(jnp.int32, (bc, bm), 0)
    ).astype(jnp.bfloat16)
    out = lax.dot_general(
        s, r2.astype(jnp.bfloat16), (((1,), (0,)), ((), ())),
        preferred_element_type=jnp.float32,
    )
    if add_bias:
        out = out + b_ref[...]
    o_ref[...] = out


def kernel(cellgene_embedding, genes_oi, weight1, bias1):
    n_cells, n_genes_oi, n_in = cellgene_embedding.shape
    n_out = weight1.shape[2]

    info = plsc.get_sparse_core_info()
    num_workers = info.num_cores * info.num_subcores
    rows_per_worker = n_genes_oi // num_workers

    x_perm = jnp.transpose(cellgene_embedding, (0, 2, 1))
    bias2 = bias1.reshape(1, n_out)
    genes32 = genes_oi.astype(jnp.int32)

    d = n_in * n_out
    bc = 64
    grid = (n_cells // bc,)

    table_pk = jax.lax.bitcast_convert_type(
        weight1.reshape(weight1.shape[0], d // 2, 2).astype(jnp.bfloat16),
        jnp.int32,
    )
    gather = _make_sc_gather(n_genes_oi, d // 2, rows_per_worker,
                             info.num_cores)
    wr_pk = gather(table_pk, genes32)
    wr = jax.lax.bitcast_convert_type(wr_pk, jnp.bfloat16).reshape(
        n_genes_oi, d
    )

    body = functools.partial(_matmul_body, n_in, n_out, True)
    out = pl.pallas_call(
        body,
        grid=grid,
        in_specs=[
            pl.BlockSpec((bc, n_in, n_genes_oi), lambda k: (k, 0, 0)),
            pl.BlockSpec((n_genes_oi, d), lambda k: (0, 0)),
            pl.BlockSpec((1, n_out), lambda k: (0, 0)),
        ],
        out_specs=pl.BlockSpec((bc, n_out), lambda k: (k, 0)),
        out_shape=jax.ShapeDtypeStruct((n_cells, n_out), jnp.float32),
        compiler_params=pltpu.CompilerParams(
            dimension_semantics=("arbitrary",),
        ),
    )(x_perm, wr, bias2)
    return out


# speedup vs baseline: 2.9114x; 2.9114x over previous
import functools

import jax
import jax.numpy as jnp
from jax import lax
from jax.experimental import pallas as pl
from jax.experimental.pallas import tpu as pltpu
from jax.experimental.pallas import tpu_sc as plsc


def _make_sc_gather(n_rows, d, rows_per_worker, num_cores):
    def body(table_hbm, idx_hbm, out_hbm, idx_v, rows_v, sem):
        wid = lax.axis_index("s") * num_cores + lax.axis_index("c")
        base = wid * rows_per_worker
        pltpu.sync_copy(idx_hbm.at[pl.ds(base, rows_per_worker)], idx_v)
        pltpu.async_copy(table_hbm.at[idx_v], rows_v, sem).wait()
        pltpu.sync_copy(rows_v, out_hbm.at[pl.ds(base, rows_per_worker)])

    return pl.kernel(
        body,
        out_type=jax.ShapeDtypeStruct((n_rows, d), jnp.float32),
        mesh=plsc.VectorSubcoreMesh(core_axis_name="c", subcore_axis_name="s"),
        scratch_types=[
            pltpu.VMEM((rows_per_worker,), jnp.int32),
            pltpu.VMEM((rows_per_worker, d), jnp.float32),
            pltpu.SemaphoreType.DMA,
        ],
    )


def _matmul_body(csz, n_out, add_bias, x_ref, w_ref, b_ref, o_ref):
    bc = x_ref.shape[0]
    bm = bc * csz
    nc = csz * n_out
    x3 = x_ref[...]
    xb = x3.reshape(bm, x3.shape[2]).astype(jnp.bfloat16)
    wb = w_ref[...].astype(jnp.bfloat16)
    p = lax.dot_general(
        xb, wb, (((1,), (0,)), ((), ())), preferred_element_type=jnp.float32
    )
    rows = lax.broadcasted_iota(jnp.int32, p.shape, 0)
    lanes = lax.broadcasted_iota(jnp.int32, p.shape, 1)
    z = jnp.where((rows % csz) == (lanes // n_out), p, 0.0).astype(jnp.bfloat16)
    f = (
        lax.broadcasted_iota(jnp.int32, (nc, n_out), 0) % n_out
        == lax.broadcasted_iota(jnp.int32, (nc, n_out), 1)
    ).astype(jnp.bfloat16)
    r2 = lax.dot_general(
        z, f, (((1,), (0,)), ((), ())), preferred_element_type=jnp.float32
    )
    s = (
        lax.broadcasted_iota(jnp.int32, (bc, bm), 1) // csz
        == lax.broadcasted_iota(jnp.int32, (bc, bm), 0)
    ).astype(jnp.bfloat16)
    out = lax.dot_general(
        s, r2.astype(jnp.bfloat16), (((1,), (0,)), ((), ())),
        preferred_element_type=jnp.float32,
    )
    if add_bias:
        out = out + b_ref[...]
    o_ref[...] = out


def kernel(cellgene_embedding, genes_oi, weight1, bias1):
    n_cells, n_genes_oi, n_in = cellgene_embedding.shape
    n_out = weight1.shape[2]

    info = plsc.get_sparse_core_info()
    num_workers = info.num_cores * info.num_subcores
    rows_per_worker = n_genes_oi // num_workers

    x_perm = jnp.transpose(cellgene_embedding, (0, 2, 1))
    bias2 = bias1.reshape(1, n_out)
    genes32 = genes_oi.astype(jnp.int32)

    d = n_in * n_out
    bc = 64
    grid = (n_cells // bc,)

    gather = _make_sc_gather(n_genes_oi, d, rows_per_worker, info.num_cores)
    table2d = weight1.reshape(weight1.shape[0], d)
    wr = gather(table2d, genes32)

    body = functools.partial(_matmul_body, n_in, n_out, True)
    out = pl.pallas_call(
        body,
        grid=grid,
        in_specs=[
            pl.BlockSpec((bc, n_in, n_genes_oi), lambda k: (k, 0, 0)),
            pl.BlockSpec((n_genes_oi, d), lambda k: (0, 0)),
            pl.BlockSpec((1, n_out), lambda k: (0, 0)),
        ],
        out_specs=pl.BlockSpec((bc, n_out), lambda k: (k, 0)),
        out_shape=jax.ShapeDtypeStruct((n_cells, n_out), jnp.float32),
        compiler_params=pltpu.CompilerParams(
            dimension_semantics=("arbitrary",),
        ),
    )(x_perm, wr, bias2)
    return out
